# baseline (speedup 1.0000x reference)
import functools

import jax
import jax.numpy as jnp
from jax import lax
from jax.experimental import pallas as pl
from jax.experimental.pallas import tpu as pltpu
from jax.experimental.pallas import tpu_sc as plsc

BATCH = 4096
DIM = 4096
NUM_CORES = 2
NUM_SUBCORES = 16
NUM_WORKERS = NUM_CORES * NUM_SUBCORES
BPW = BATCH // NUM_WORKERS
CHUNK = 8
NCHUNK = BPW // CHUNK
NBUF = 3
STEADY = (NCHUNK - NBUF) // NBUF * NBUF

_MESH = plsc.VectorSubcoreMesh(
    core_axis_name="c", subcore_axis_name="s",
    num_cores=NUM_CORES, num_subcores=NUM_SUBCORES)


@functools.partial(
    pl.kernel,
    out_type=(
        jax.ShapeDtypeStruct((BATCH, DIM), jnp.float32),
        jax.ShapeDtypeStruct((BATCH, DIM), jnp.float32),
    ),
    mesh=_MESH,
    scratch_types=[
        pltpu.VMEM((BPW,), jnp.int32),
        pltpu.VMEM((BPW,), jnp.int32),
        [pltpu.VMEM((CHUNK, DIM), jnp.float32)] * NBUF,
        [pltpu.SemaphoreType.DMA] * NBUF,
        [pltpu.SemaphoreType.DMA] * NBUF,
    ],
)
def _gather2(users_hbm, items_hbm, u_tab, i_tab, u_out, i_out,
             uidx, iidx, bufs, gsems, ssems):
    wid = lax.axis_index("s") * NUM_CORES + lax.axis_index("c")
    base = wid * BPW
    pltpu.sync_copy(users_hbm.at[pl.ds(base, BPW)], uidx)
    pltpu.sync_copy(items_hbm.at[pl.ds(base, BPW)], iidx)

    def run(tab, idx, out):
        def gstart(c, b):
            pltpu.async_copy(tab.at[idx.at[pl.ds(c * CHUNK, CHUNK)]],
                             bufs[b], gsems[b])

        def gwait(b):
            pltpu.make_async_copy(tab.at[pl.ds(0, CHUNK)], bufs[b],
                                  gsems[b]).wait()

        def sstart(c, b):
            pltpu.async_copy(bufs[b], out.at[pl.ds(base + c * CHUNK, CHUNK)],
                             ssems[b])

        def swait(b):
            pltpu.make_async_copy(bufs[b], out.at[pl.ds(base, CHUNK)],
                                  ssems[b]).wait()

        for b in range(NBUF):
            gstart(b, b)

        @pl.loop(0, STEADY, step=NBUF)
        def _steady(c):
            for b in range(NBUF):
                gwait(b)
                sstart(c + b, b)
            for b in range(NBUF):
                swait(b)
                gstart(c + NBUF + b, b)

        for c in range(STEADY, NCHUNK):
            b = c % NBUF
            if c >= STEADY + NBUF:
                swait(b)
                gstart(c, b)
            gwait(b)
            sstart(c, b)
        for b in range(NBUF):
            swait(b)

    run(u_tab, uidx, u_out)
    run(i_tab, iidx, i_out)


def kernel(users, items, user_table, item_table):
    u_repr, i_repr = _gather2(users, items, user_table, item_table)
    return (u_repr, i_repr)
